# Initial kernel scaffold; baseline (speedup 1.0000x reference)
import functools

import jax
import jax.numpy as jnp
from jax import lax
from jax.experimental import pallas as pl
from jax.experimental.pallas import tpu as pltpu
from jax.experimental.pallas import tpu_sc as plsc

N_USERS = 25000
N_ITEMS = 25000
N_NODES = 50000
D = 64
E = 800000
N_LAYERS = 3

NC, NS = 2, 16
NW = NC * NS

HALF = 25000
HPAD = 25088
NPAD = 2 * HPAD
ROWS_PER_TILE = HPAD // NS
DUMP = HPAD
ZROWS = HPAD + 16
PAD_OFF = HPAD - HALF

EP = 819200
EDGES_PER_TILE = EP // NW
CHUNK = 128
CHUNKS_PER_TILE = EDGES_PER_TILE // CHUNK

_mesh = plsc.VectorSubcoreMesh(core_axis_name="c", subcore_axis_name="s")


def _sc_degree(row_p, ones16, zeros16):
    @functools.partial(
        pl.kernel,
        out_type=jax.ShapeDtypeStruct((NPAD, 16), jnp.float32),
        mesh=_mesh,
        scratch_types=[
            pltpu.VMEM((CHUNK,), jnp.int32),
            pltpu.VMEM((CHUNK,), jnp.int32),
            pltpu.VMEM((CHUNK, 16), jnp.float32),
            pltpu.VMEM_SHARED((ZROWS, 16), jnp.float32),
        ],
    )
    def k(row_hbm, ones_hbm, zeros_hbm, deg_hbm, row_v, sidx_v, ones_v, deg_sp):
        c = lax.axis_index("c")
        s = lax.axis_index("s")
        wid = c * NS + s
        base_node = c * HALF

        pltpu.sync_copy(
            zeros_hbm.at[pl.ds(s * ROWS_PER_TILE, ROWS_PER_TILE)],
            deg_sp.at[pl.ds(s * ROWS_PER_TILE, ROWS_PER_TILE)],
        )
        pltpu.sync_copy(ones_hbm, ones_v)
        plsc.subcore_barrier()

        ebase = wid * EDGES_PER_TILE

        @pl.loop(0, CHUNKS_PER_TILE)
        def _(kk):
            pltpu.sync_copy(row_hbm.at[pl.ds(ebase + kk * CHUNK, CHUNK)], row_v)
            for g in range(CHUNK // 16):
                sl = pl.ds(g * 16, 16)
                sloc = row_v[sl] - base_node
                ok = (sloc >= 0) & (sloc < HALF)
                sidx_v[sl] = jnp.where(ok, sloc, DUMP)
            pltpu.sync_copy(ones_v, deg_sp.at[sidx_v], add=True)

        plsc.subcore_barrier()
        pltpu.sync_copy(
            deg_sp.at[pl.ds(s * ROWS_PER_TILE, ROWS_PER_TILE)],
            deg_hbm.at[pl.ds(c * HPAD + s * ROWS_PER_TILE, ROWS_PER_TILE)],
        )

    return k(row_p, ones16, zeros16)


def _sc_spmm(y, row_p, col_p, zeros64):
    @functools.partial(
        pl.kernel,
        out_type=jax.ShapeDtypeStruct((NPAD, D), jnp.float32),
        mesh=_mesh,
        scratch_types=[
            pltpu.VMEM((CHUNK,), jnp.int32),
            pltpu.VMEM((CHUNK,), jnp.int32),
            pltpu.VMEM((CHUNK,), jnp.int32),
            pltpu.VMEM((CHUNK,), jnp.int32),
            pltpu.VMEM((CHUNK, D), jnp.float32),
            pltpu.VMEM_SHARED((ZROWS, D), jnp.float32),
            pltpu.SemaphoreType.DMA,
        ],
    )
    def k(y_hbm, row_hbm, col_hbm, zeros_hbm, z_hbm,
          row_v, col_v, sidx_v, gidx_v, rows_v, z_sp, sem):
        c = lax.axis_index("c")
        s = lax.axis_index("s")
        wid = c * NS + s
        base_node = c * HALF

        pltpu.sync_copy(
            zeros_hbm.at[pl.ds(s * ROWS_PER_TILE, ROWS_PER_TILE)],
            z_sp.at[pl.ds(s * ROWS_PER_TILE, ROWS_PER_TILE)],
        )
        plsc.subcore_barrier()

        ebase = wid * EDGES_PER_TILE

        @pl.loop(0, CHUNKS_PER_TILE)
        def _(kk):
            off = ebase + kk * CHUNK
            pltpu.sync_copy(row_hbm.at[pl.ds(off, CHUNK)], row_v)
            pltpu.sync_copy(col_hbm.at[pl.ds(off, CHUNK)], col_v)
            for g in range(CHUNK // 16):
                sl = pl.ds(g * 16, 16)
                sloc = row_v[sl] - base_node
                ok = (sloc >= 0) & (sloc < HALF)
                sidx_v[sl] = jnp.where(ok, sloc, DUMP)
                cc = col_v[sl]
                gidx_v[sl] = jnp.where(cc >= HALF, cc + PAD_OFF, cc)
            pltpu.async_copy(y_hbm.at[gidx_v], rows_v, sem).wait()
            pltpu.sync_copy(rows_v, z_sp.at[sidx_v], add=True)

        plsc.subcore_barrier()
        pltpu.sync_copy(
            z_sp.at[pl.ds(s * ROWS_PER_TILE, ROWS_PER_TILE)],
            z_hbm.at[pl.ds(c * HPAD + s * ROWS_PER_TILE, ROWS_PER_TILE)],
        )

    return k(y, row_p, col_p, zeros64)


_TC_R = 512


def _tc_prep(deg, x0):
    def body(deg_ref, x_ref, dinv_ref, y_ref, acc_ref):
        dg = deg_ref[...]
        dinv = jnp.where(dg > 0.0, lax.rsqrt(dg), 0.0)
        dinv_ref[...] = dinv
        x = x_ref[...]
        y_ref[...] = x * dinv[:, 0:1]
        acc_ref[...] = 0.25 * x

    return pl.pallas_call(
        body,
        grid=(NPAD // _TC_R,),
        in_specs=[
            pl.BlockSpec((_TC_R, 16), lambda i: (i, 0)),
            pl.BlockSpec((_TC_R, D), lambda i: (i, 0)),
        ],
        out_specs=[
            pl.BlockSpec((_TC_R, 16), lambda i: (i, 0)),
            pl.BlockSpec((_TC_R, D), lambda i: (i, 0)),
            pl.BlockSpec((_TC_R, D), lambda i: (i, 0)),
        ],
        out_shape=[
            jax.ShapeDtypeStruct((NPAD, 16), jnp.float32),
            jax.ShapeDtypeStruct((NPAD, D), jnp.float32),
            jax.ShapeDtypeStruct((NPAD, D), jnp.float32),
        ],
    )(deg, x0)


def _tc_update(z, dinv, acc, last):
    def body_mid(z_ref, dinv_ref, acc_ref, acc_out, y_out):
        d1 = dinv_ref[...][:, 0:1]
        xn = z_ref[...] * d1
        acc_out[...] = acc_ref[...] + 0.25 * xn
        y_out[...] = xn * d1

    def body_last(z_ref, dinv_ref, acc_ref, acc_out):
        d1 = dinv_ref[...][:, 0:1]
        xn = z_ref[...] * d1
        acc_out[...] = acc_ref[...] + 0.25 * xn

    n_out = 1 if last else 2
    return pl.pallas_call(
        body_last if last else body_mid,
        grid=(NPAD // _TC_R,),
        in_specs=[
            pl.BlockSpec((_TC_R, D), lambda i: (i, 0)),
            pl.BlockSpec((_TC_R, 16), lambda i: (i, 0)),
            pl.BlockSpec((_TC_R, D), lambda i: (i, 0)),
        ],
        out_specs=[pl.BlockSpec((_TC_R, D), lambda i: (i, 0))] * n_out,
        out_shape=[jax.ShapeDtypeStruct((NPAD, D), jnp.float32)] * n_out,
    )(z, dinv, acc)


def kernel(user_emb, item_emb, edge_index):
    row = edge_index[0]
    col = edge_index[1]
    row_p = jnp.concatenate([row, jnp.full((EP - E,), 2**30, jnp.int32)])
    col_p = jnp.concatenate([col, jnp.zeros((EP - E,), jnp.int32)])

    x0 = jnp.zeros((NPAD, D), jnp.float32)
    x0 = x0.at[0:N_USERS].set(user_emb)
    x0 = x0.at[HPAD:HPAD + N_ITEMS].set(item_emb)

    ones16 = jnp.ones((CHUNK, 16), jnp.float32)
    zeros16 = jnp.zeros((HPAD, 16), jnp.float32)
    zeros64 = jnp.zeros((HPAD, D), jnp.float32)

    deg = _sc_degree(row_p, ones16, zeros16)
    dinv, y, acc = _tc_prep(deg, x0)
    for layer in range(N_LAYERS):
        z = _sc_spmm(y, row_p, col_p, zeros64)
        if layer < N_LAYERS - 1:
            acc, y = _tc_update(z, dinv, acc, last=False)
        else:
            (acc,) = _tc_update(z, dinv, acc, last=True)

    return acc[0:N_USERS], acc[HPAD:HPAD + N_ITEMS]


# speedup vs baseline: 4.3989x; 4.3989x over previous
import functools

import jax
import jax.numpy as jnp
from jax import lax
from jax.experimental import pallas as pl
from jax.experimental.pallas import tpu as pltpu
from jax.experimental.pallas import tpu_sc as plsc

N_USERS = 25000
N_ITEMS = 25000
N_NODES = 50000
D = 64
E = 800000
N_LAYERS = 3

NC, NS = 2, 16
NW = NC * NS

HALF = 25000
HPAD = 25088
NPAD = 2 * HPAD
ROWS_PER_TILE = HPAD // NS
DUMP = HPAD
ZROWS = HPAD + 16
PAD_OFF = HPAD - HALF

EP = 819200
EDGES_PER_TILE = EP // NS
CHUNK = 128
CHUNKS_PER_TILE = EDGES_PER_TILE // CHUNK

_mesh = plsc.VectorSubcoreMesh(core_axis_name="c", subcore_axis_name="s")
_sc_params = pltpu.CompilerParams(use_tc_tiling_on_sc=False)


def _sc_degree(row_p, ones16, zeros16):
    @functools.partial(
        pl.kernel,
        out_type=jax.ShapeDtypeStruct((NPAD, 16), jnp.float32),
        mesh=_mesh,
        compiler_params=_sc_params,
        scratch_types=[
            pltpu.VMEM((CHUNK,), jnp.int32),
            pltpu.VMEM((CHUNK,), jnp.int32),
            pltpu.VMEM((CHUNK, 16), jnp.float32),
            pltpu.VMEM_SHARED((ZROWS, 16), jnp.float32),
        ],
    )
    def k(row_hbm, ones_hbm, zeros_hbm, deg_hbm, row_v, sidx_v, ones_v, deg_sp):
        c = lax.axis_index("c")
        s = lax.axis_index("s")
        base_node = c * HALF

        pltpu.sync_copy(
            zeros_hbm.at[pl.ds(s * ROWS_PER_TILE, ROWS_PER_TILE)],
            deg_sp.at[pl.ds(s * ROWS_PER_TILE, ROWS_PER_TILE)],
        )
        pltpu.sync_copy(ones_hbm, ones_v)
        plsc.subcore_barrier()

        ebase = s * EDGES_PER_TILE

        @pl.loop(0, CHUNKS_PER_TILE)
        def _(kk):
            pltpu.sync_copy(row_hbm.at[pl.ds(ebase + kk * CHUNK, CHUNK)], row_v)
            for g in range(CHUNK // 16):
                sl = pl.ds(g * 16, 16)
                sloc = row_v[sl] - base_node
                ok = (sloc >= 0) & (sloc < HALF)
                sidx_v[sl] = jnp.where(ok, sloc, DUMP)
            pltpu.sync_copy(ones_v, deg_sp.at[sidx_v], add=True)

        plsc.subcore_barrier()
        pltpu.sync_copy(
            deg_sp.at[pl.ds(s * ROWS_PER_TILE, ROWS_PER_TILE)],
            deg_hbm.at[pl.ds(c * HPAD + s * ROWS_PER_TILE, ROWS_PER_TILE)],
        )

    return k(row_p, ones16, zeros16)


def _sc_spmm(y, row_p, col_p, zeros64):
    @functools.partial(
        pl.kernel,
        out_type=jax.ShapeDtypeStruct((NPAD, D), jnp.float32),
        mesh=_mesh,
        compiler_params=_sc_params,
        scratch_types=[
            pltpu.VMEM((CHUNK,), jnp.int32),
            pltpu.VMEM((CHUNK,), jnp.int32),
            pltpu.VMEM((CHUNK,), jnp.int32),
            pltpu.VMEM((CHUNK,), jnp.int32),
            pltpu.VMEM((CHUNK, D), jnp.float32),
            pltpu.VMEM_SHARED((ZROWS, D), jnp.float32),
            pltpu.SemaphoreType.DMA,
        ],
    )
    def k(y_hbm, row_hbm, col_hbm, zeros_hbm, z_hbm,
          row_v, col_v, sidx_v, gidx_v, rows_v, z_sp, sem):
        c = lax.axis_index("c")
        s = lax.axis_index("s")
        base_node = c * HALF

        pltpu.sync_copy(
            zeros_hbm.at[pl.ds(s * ROWS_PER_TILE, ROWS_PER_TILE)],
            z_sp.at[pl.ds(s * ROWS_PER_TILE, ROWS_PER_TILE)],
        )
        plsc.subcore_barrier()

        ebase = s * EDGES_PER_TILE

        @pl.loop(0, CHUNKS_PER_TILE)
        def _(kk):
            off = ebase + kk * CHUNK
            pltpu.sync_copy(row_hbm.at[pl.ds(off, CHUNK)], row_v)
            pltpu.sync_copy(col_hbm.at[pl.ds(off, CHUNK)], col_v)
            for g in range(CHUNK // 16):
                sl = pl.ds(g * 16, 16)
                sloc = row_v[sl] - base_node
                ok = (sloc >= 0) & (sloc < HALF)
                sidx_v[sl] = jnp.where(ok, sloc, DUMP)
                cc = col_v[sl]
                gidx_v[sl] = jnp.where(cc >= HALF, cc + PAD_OFF, cc)
            pltpu.async_copy(y_hbm.at[gidx_v], rows_v, sem).wait()
            pltpu.sync_copy(rows_v, z_sp.at[sidx_v], add=True)

        plsc.subcore_barrier()
        pltpu.sync_copy(
            z_sp.at[pl.ds(s * ROWS_PER_TILE, ROWS_PER_TILE)],
            z_hbm.at[pl.ds(c * HPAD + s * ROWS_PER_TILE, ROWS_PER_TILE)],
        )

    return k(y, row_p, col_p, zeros64)


_TC_R = 512


def _tc_prep(deg, x0):
    def body(deg_ref, x_ref, dinv_ref, y_ref, acc_ref):
        dg = deg_ref[...]
        dinv = jnp.where(dg > 0.0, lax.rsqrt(dg), 0.0)
        dinv_ref[...] = dinv
        x = x_ref[...]
        y_ref[...] = x * dinv[:, 0:1]
        acc_ref[...] = 0.25 * x

    return pl.pallas_call(
        body,
        grid=(NPAD // _TC_R,),
        in_specs=[
            pl.BlockSpec((_TC_R, 16), lambda i: (i, 0)),
            pl.BlockSpec((_TC_R, D), lambda i: (i, 0)),
        ],
        out_specs=[
            pl.BlockSpec((_TC_R, 16), lambda i: (i, 0)),
            pl.BlockSpec((_TC_R, D), lambda i: (i, 0)),
            pl.BlockSpec((_TC_R, D), lambda i: (i, 0)),
        ],
        out_shape=[
            jax.ShapeDtypeStruct((NPAD, 16), jnp.float32),
            jax.ShapeDtypeStruct((NPAD, D), jnp.float32),
            jax.ShapeDtypeStruct((NPAD, D), jnp.float32),
        ],
    )(deg, x0)


def _tc_update(z, dinv, acc, last):
    def body_mid(z_ref, dinv_ref, acc_ref, acc_out, y_out):
        d1 = dinv_ref[...][:, 0:1]
        xn = z_ref[...] * d1
        acc_out[...] = acc_ref[...] + 0.25 * xn
        y_out[...] = xn * d1

    def body_last(z_ref, dinv_ref, acc_ref, acc_out):
        d1 = dinv_ref[...][:, 0:1]
        xn = z_ref[...] * d1
        acc_out[...] = acc_ref[...] + 0.25 * xn

    n_out = 1 if last else 2
    return pl.pallas_call(
        body_last if last else body_mid,
        grid=(NPAD // _TC_R,),
        in_specs=[
            pl.BlockSpec((_TC_R, D), lambda i: (i, 0)),
            pl.BlockSpec((_TC_R, 16), lambda i: (i, 0)),
            pl.BlockSpec((_TC_R, D), lambda i: (i, 0)),
        ],
        out_specs=[pl.BlockSpec((_TC_R, D), lambda i: (i, 0))] * n_out,
        out_shape=[jax.ShapeDtypeStruct((NPAD, D), jnp.float32)] * n_out,
    )(z, dinv, acc)


def kernel(user_emb, item_emb, edge_index):
    row = edge_index[0]
    col = edge_index[1]
    row_p = jnp.concatenate([row, jnp.full((EP - E,), 2**30, jnp.int32)])
    col_p = jnp.concatenate([col, jnp.zeros((EP - E,), jnp.int32)])

    x0 = jnp.zeros((NPAD, D), jnp.float32)
    x0 = x0.at[0:N_USERS].set(user_emb)
    x0 = x0.at[HPAD:HPAD + N_ITEMS].set(item_emb)

    ones16 = jnp.ones((CHUNK, 16), jnp.float32)
    zeros16 = jnp.zeros((HPAD, 16), jnp.float32)
    zeros64 = jnp.zeros((HPAD, D), jnp.float32)

    deg = _sc_degree(row_p, ones16, zeros16)
    dinv, y, acc = _tc_prep(deg, x0)
    for layer in range(N_LAYERS):
        z = _sc_spmm(y, row_p, col_p, zeros64)
        if layer < N_LAYERS - 1:
            acc, y = _tc_update(z, dinv, acc, last=False)
        else:
            (acc,) = _tc_update(z, dinv, acc, last=True)

    return acc[0:N_USERS], acc[HPAD:HPAD + N_ITEMS]
